# baseline (speedup 1.0000x reference)
import functools

import jax
import jax.numpy as jnp
from jax import lax
from jax.experimental import pallas as pl
from jax.experimental.pallas import tpu as pltpu
from jax.experimental.pallas import tpu_sc as plsc

NC = 2
NS = 16
NW = NC * NS
C = 80
NP = 10240
NPA = 10112


def _mesh():
    return plsc.VectorSubcoreMesh(core_axis_name="c", subcore_axis_name="s")


def _fill(ref, n, value):
    vec = jnp.full((16,), value, jnp.float32)

    def body(i, carry):
        ref[pl.ds(i * 16, 16)] = vec
        return carry

    lax.fori_loop(0, n // 16, body, None)


@functools.partial(jax.jit, static_argnums=(2, 3))
def _degrees(src2d, dst2d, E, n_pad):
    chunks = E // NW // C
    seg = n_pad // NS

    @functools.partial(
        pl.kernel,
        out_type=(
            jax.ShapeDtypeStruct((NC, n_pad), jnp.float32),
            jax.ShapeDtypeStruct((NC, n_pad), jnp.float32),
        ),
        mesh=_mesh(),
        scratch_types=[
            pltpu.VMEM((chunks, C), jnp.int32),
            pltpu.VMEM((chunks, C), jnp.int32),
            pltpu.VMEM((C,), jnp.float32),
            pltpu.VMEM((seg,), jnp.float32),
            pltpu.VMEM_SHARED((n_pad,), jnp.float32),
            pltpu.VMEM_SHARED((n_pad,), jnp.float32),
            pltpu.SemaphoreType.DMA,
            pltpu.SemaphoreType.DMA,
        ],
    )
    def deg_kernel(src_hbm, dst_hbm, out_s, out_d,
                   idx_s, idx_d, ones_v, zeros_v, hist_s, hist_d,
                   sem_s, sem_d):
        cid = lax.axis_index("c")
        sid = lax.axis_index("s")
        wid = cid * NS + sid

        _fill(ones_v, C, 1.0)
        _fill(zeros_v, seg, 0.0)

        pltpu.sync_copy(src_hbm.at[wid], idx_s)
        pltpu.sync_copy(dst_hbm.at[wid], idx_d)

        pltpu.sync_copy(zeros_v, hist_s.at[pl.ds(sid * seg, seg)])
        pltpu.sync_copy(zeros_v, hist_d.at[pl.ds(sid * seg, seg)])
        plsc.subcore_barrier()

        def s_start(j):
            pltpu.make_async_copy(ones_v, hist_s.at[idx_s.at[j]],
                                  sem_s).start(add=True)
            pltpu.make_async_copy(ones_v, hist_d.at[idx_d.at[j]],
                                  sem_d).start(add=True)

        def s_wait(j):
            pltpu.make_async_copy(ones_v, hist_s.at[idx_s.at[j]],
                                  sem_s).wait()
            pltpu.make_async_copy(ones_v, hist_d.at[idx_d.at[j]],
                                  sem_d).wait()

        def body(j, carry):
            s_start(j)

            @pl.when(j >= 4)
            def _():
                s_wait(j - 4)

            return carry

        lax.fori_loop(0, chunks, body, None)

        def drain(j, carry):
            s_wait(j)
            return carry

        lax.fori_loop(chunks - 4, chunks, drain, None)
        plsc.subcore_barrier()

        pltpu.sync_copy(hist_s.at[pl.ds(sid * seg, seg)],
                        out_s.at[cid, pl.ds(sid * seg, seg)])
        pltpu.sync_copy(hist_d.at[pl.ds(sid * seg, seg)],
                        out_d.at[cid, pl.ds(sid * seg, seg)])

    return deg_kernel(src2d, dst2d)


NBUF = 4
NBUF_I = 8


@functools.partial(jax.jit, static_argnums=(3, 4, 5))
def _aggregate(h, src4, dst4, E, n_pad, D):
    chunks = E // NW // C
    rows = n_pad // NS

    @functools.partial(
        pl.kernel,
        out_type=jax.ShapeDtypeStruct((NC, n_pad, D), jnp.float32),
        mesh=_mesh(),
        scratch_types=[
            pltpu.VMEM((NBUF_I, 2, C), jnp.int32),
            pltpu.VMEM((NBUF, C, D), jnp.float32),
            pltpu.VMEM((40, D), jnp.float32),
            pltpu.VMEM_SHARED((n_pad, D), jnp.float32),
            pltpu.SemaphoreType.DMA,
            pltpu.SemaphoreType.DMA,
            pltpu.SemaphoreType.DMA,
            pltpu.SemaphoreType.DMA,
        ],
    )
    def agg_kernel(h_hbm, src_hbm, dst_hbm, out_hbm, idx, buf, zbuf, acc,
                   sem_i, sem_g, sem_s, sem_z):
        cid = lax.axis_index("c")
        sid = lax.axis_index("s")
        wid = cid * NS + sid

        def idx_start(j):
            pltpu.make_async_copy(src_hbm.at[wid, j, 0],
                                  idx.at[j % NBUF_I, 0], sem_i).start()
            pltpu.make_async_copy(dst_hbm.at[wid, j, 0],
                                  idx.at[j % NBUF_I, 1], sem_i).start()

        def idx_wait(j):
            pltpu.make_async_copy(src_hbm.at[wid, j, 0],
                                  idx.at[j % NBUF_I, 0], sem_i).wait()
            pltpu.make_async_copy(dst_hbm.at[wid, j, 0],
                                  idx.at[j % NBUF_I, 1], sem_i).wait()

        def gat_start(j):
            pltpu.make_async_copy(h_hbm.at[idx.at[j % NBUF_I, 0]],
                                  buf.at[j % NBUF], sem_g).start()

        def gat_wait(j):
            pltpu.make_async_copy(h_hbm.at[idx.at[j % NBUF_I, 0]],
                                  buf.at[j % NBUF], sem_g).wait()

        def scat_start(j):
            pltpu.make_async_copy(buf.at[j % NBUF],
                                  acc.at[idx.at[j % NBUF_I, 1]],
                                  sem_s).start(add=True)

        def scat_wait(j):
            pltpu.make_async_copy(buf.at[j % NBUF],
                                  acc.at[idx.at[j % NBUF_I, 1]],
                                  sem_s).wait()

        zvec = jnp.zeros((16,), jnp.float32)
        ZR = 40

        def zfill(k, carry):
            zbuf[k // (D // 16), pl.ds((k % (D // 16)) * 16, 16)] = zvec
            return carry

        lax.fori_loop(0, ZR * (D // 16), zfill, None)
        base = sid * rows
        nz = rows // ZR
        rem = rows - nz * ZR

        def z_start(i, carry):
            pltpu.make_async_copy(zbuf, acc.at[pl.ds(base + i * ZR, ZR)],
                                  sem_z).start()
            return carry

        lax.fori_loop(0, nz, z_start, None)
        if rem:
            pltpu.make_async_copy(zbuf.at[pl.ds(0, rem)],
                                  acc.at[pl.ds(base + nz * ZR, rem)],
                                  sem_z).start()

        for j in range(5):
            idx_start(j)
        idx_wait(0)
        gat_start(0)
        idx_wait(1)
        gat_start(1)

        def z_wait(i, carry):
            pltpu.make_async_copy(zbuf, acc.at[pl.ds(base + i * ZR, ZR)],
                                  sem_z).wait()
            return carry

        lax.fori_loop(0, nz, z_wait, None)
        if rem:
            pltpu.make_async_copy(zbuf.at[pl.ds(0, rem)],
                                  acc.at[pl.ds(base + nz * ZR, rem)],
                                  sem_z).wait()
        plsc.subcore_barrier()

        def body(j, carry):
            @pl.when(j >= 2)
            def _():
                scat_wait(j - 2)

            @pl.when(j + 5 < chunks)
            def _():
                idx_start(j + 5)

            @pl.when(j + 2 < chunks)
            def _():
                idx_wait(j + 2)
                gat_start(j + 2)

            gat_wait(j)
            scat_start(j)
            return carry

        lax.fori_loop(0, chunks, body, None)
        scat_wait(chunks - 2)
        scat_wait(chunks - 1)
        plsc.subcore_barrier()

        pltpu.sync_copy(acc.at[pl.ds(base, rows)],
                        out_hbm.at[cid, pl.ds(base, rows)])

    return agg_kernel(h, src4, dst4)


def _scale_body(x_ref, deg_ref, o_ref):
    deg = deg_ref[:, 0] + deg_ref[:, 1]
    norm = lax.rsqrt(jnp.maximum(deg, 1.0))
    o_ref[...] = x_ref[...] * norm[:, None]


def _out_body(p_ref, deg_ref, w_ref, b_ref, o_ref):
    deg = deg_ref[:, 0] + deg_ref[:, 1]
    norm = lax.rsqrt(jnp.maximum(deg, 1.0))
    rst = (p_ref[0] + p_ref[1]) * norm[:, None]
    o_ref[...] = (jnp.dot(rst, w_ref[...], preferred_element_type=jnp.float32)
                  + b_ref[...])


def kernel(x, edge_index, W, b):
    N, D = x.shape
    E = edge_index.shape[1]
    chunks = E // NW // C
    src2d = edge_index[0].reshape(NW, chunks, C)
    dst2d = edge_index[1].reshape(NW, chunks, C)
    src4 = edge_index[0].reshape(NW, chunks, 1, C)
    dst4 = edge_index[1].reshape(NW, chunks, 1, C)

    deg_s, deg_d = _degrees(src2d, dst2d, E, NP)
    deg_s = deg_s[:, :N].T
    deg_d = deg_d[:, :N].T

    R = 1000
    grid = (N // R,)
    h = pl.pallas_call(
        _scale_body,
        grid=grid,
        in_specs=[
            pl.BlockSpec((R, D), lambda i: (i, 0)),
            pl.BlockSpec((R, NC), lambda i: (i, 0)),
        ],
        out_specs=pl.BlockSpec((R, D), lambda i: (i, 0)),
        out_shape=jax.ShapeDtypeStruct((N, D), jnp.float32),
    )(x, deg_s)

    parts = _aggregate(h, src4, dst4, E, NPA, D)

    out = pl.pallas_call(
        _out_body,
        grid=grid,
        in_specs=[
            pl.BlockSpec((NC, R, D), lambda i: (0, i, 0)),
            pl.BlockSpec((R, NC), lambda i: (i, 0)),
            pl.BlockSpec((D, D), lambda i: (0, 0)),
            pl.BlockSpec((1, D), lambda i: (0, 0)),
        ],
        out_specs=pl.BlockSpec((R, D), lambda i: (i, 0)),
        out_shape=jax.ShapeDtypeStruct((N, D), jnp.float32),
    )(parts, deg_d, W, b[None, :])
    return out


# speedup vs baseline: 1.0543x; 1.0543x over previous
import functools

import jax
import jax.numpy as jnp
from jax import lax
from jax.experimental import pallas as pl
from jax.experimental.pallas import tpu as pltpu
from jax.experimental.pallas import tpu_sc as plsc

NC = 2
NS = 16
NW = NC * NS
C = 80
NP = 10240
NPA = 10112


def _mesh():
    return plsc.VectorSubcoreMesh(core_axis_name="c", subcore_axis_name="s")


def _fill(ref, n, value):
    vec = jnp.full((16,), value, jnp.float32)

    def body(i, carry):
        ref[pl.ds(i * 16, 16)] = vec
        return carry

    lax.fori_loop(0, n // 16, body, None)


@functools.partial(jax.jit, static_argnums=(2, 3))
def _degrees(src2d, dst2d, E, n_pad):
    chunks = E // NW // C
    seg = n_pad // NS

    @functools.partial(
        pl.kernel,
        out_type=(
            jax.ShapeDtypeStruct((NC, n_pad), jnp.float32),
            jax.ShapeDtypeStruct((NC, n_pad), jnp.float32),
        ),
        mesh=_mesh(),
        scratch_types=[
            pltpu.VMEM((chunks, C), jnp.int32),
            pltpu.VMEM((chunks, C), jnp.int32),
            pltpu.VMEM((C,), jnp.float32),
            pltpu.VMEM((seg,), jnp.float32),
            pltpu.VMEM_SHARED((n_pad,), jnp.float32),
            pltpu.VMEM_SHARED((n_pad,), jnp.float32),
            pltpu.SemaphoreType.DMA,
            pltpu.SemaphoreType.DMA,
        ],
    )
    def deg_kernel(src_hbm, dst_hbm, out_s, out_d,
                   idx_s, idx_d, ones_v, zeros_v, hist_s, hist_d,
                   sem_s, sem_d):
        cid = lax.axis_index("c")
        sid = lax.axis_index("s")
        wid = cid * NS + sid

        _fill(ones_v, C, 1.0)
        _fill(zeros_v, seg, 0.0)

        pltpu.sync_copy(src_hbm.at[wid], idx_s)
        pltpu.sync_copy(dst_hbm.at[wid], idx_d)

        pltpu.sync_copy(zeros_v, hist_s.at[pl.ds(sid * seg, seg)])
        pltpu.sync_copy(zeros_v, hist_d.at[pl.ds(sid * seg, seg)])
        plsc.subcore_barrier()

        def s_start(j):
            pltpu.make_async_copy(ones_v, hist_s.at[idx_s.at[j]],
                                  sem_s).start(add=True)
            pltpu.make_async_copy(ones_v, hist_d.at[idx_d.at[j]],
                                  sem_d).start(add=True)

        def s_wait(j):
            pltpu.make_async_copy(ones_v, hist_s.at[idx_s.at[j]],
                                  sem_s).wait()
            pltpu.make_async_copy(ones_v, hist_d.at[idx_d.at[j]],
                                  sem_d).wait()

        def body(j, carry):
            s_start(j)

            @pl.when(j >= 4)
            def _():
                s_wait(j - 4)

            return carry

        lax.fori_loop(0, chunks, body, None)

        def drain(j, carry):
            s_wait(j)
            return carry

        lax.fori_loop(chunks - 4, chunks, drain, None)
        plsc.subcore_barrier()

        pltpu.sync_copy(hist_s.at[pl.ds(sid * seg, seg)],
                        out_s.at[cid, pl.ds(sid * seg, seg)])
        pltpu.sync_copy(hist_d.at[pl.ds(sid * seg, seg)],
                        out_d.at[cid, pl.ds(sid * seg, seg)])

    return deg_kernel(src2d, dst2d)


NBUF = 4
NBUF_I = 8


@functools.partial(jax.jit, static_argnums=(3, 4, 5))
def _aggregate(h, src4, dst4, E, n_pad, D):
    chunks = E // NW // C
    rows = n_pad // NS

    @functools.partial(
        pl.kernel,
        out_type=jax.ShapeDtypeStruct((NC, n_pad, D), jnp.float32),
        mesh=_mesh(),
        scratch_types=[
            pltpu.VMEM((NBUF_I, 2, C), jnp.int32),
            pltpu.VMEM((NBUF, C, D), jnp.float32),
            pltpu.VMEM((40, D), jnp.float32),
            pltpu.VMEM_SHARED((n_pad, D), jnp.float32),
            pltpu.SemaphoreType.DMA,
            pltpu.SemaphoreType.DMA,
            pltpu.SemaphoreType.DMA,
            pltpu.SemaphoreType.DMA,
        ],
    )
    def agg_kernel(h_hbm, src_hbm, dst_hbm, out_hbm, idx, buf, zbuf, acc,
                   sem_i, sem_g, sem_s, sem_z):
        cid = lax.axis_index("c")
        sid = lax.axis_index("s")
        wid = cid * NS + sid

        def idx_start(j):
            pltpu.make_async_copy(src_hbm.at[wid, j, 0],
                                  idx.at[j % NBUF_I, 0], sem_i).start()
            pltpu.make_async_copy(dst_hbm.at[wid, j, 0],
                                  idx.at[j % NBUF_I, 1], sem_i).start()

        def idx_wait(j):
            pltpu.make_async_copy(src_hbm.at[wid, j, 0],
                                  idx.at[j % NBUF_I, 0], sem_i).wait()
            pltpu.make_async_copy(dst_hbm.at[wid, j, 0],
                                  idx.at[j % NBUF_I, 1], sem_i).wait()

        def gat_start(j):
            pltpu.make_async_copy(h_hbm.at[idx.at[j % NBUF_I, 0]],
                                  buf.at[j % NBUF], sem_g).start()

        def gat_wait(j):
            pltpu.make_async_copy(h_hbm.at[idx.at[j % NBUF_I, 0]],
                                  buf.at[j % NBUF], sem_g).wait()

        def scat_start(j):
            pltpu.make_async_copy(buf.at[j % NBUF],
                                  acc.at[idx.at[j % NBUF_I, 1]],
                                  sem_s).start(add=True)

        def scat_wait(j):
            pltpu.make_async_copy(buf.at[j % NBUF],
                                  acc.at[idx.at[j % NBUF_I, 1]],
                                  sem_s).wait()

        zvec = jnp.zeros((16,), jnp.float32)
        ZR = 40

        def zfill(k, carry):
            zbuf[k // (D // 16), pl.ds((k % (D // 16)) * 16, 16)] = zvec
            return carry

        lax.fori_loop(0, ZR * (D // 16), zfill, None)
        base = sid * rows
        nz = rows // ZR
        rem = rows - nz * ZR

        def z_start(i, carry):
            pltpu.make_async_copy(zbuf, acc.at[pl.ds(base + i * ZR, ZR)],
                                  sem_z).start()
            return carry

        lax.fori_loop(0, nz, z_start, None)
        if rem:
            pltpu.make_async_copy(zbuf.at[pl.ds(0, rem)],
                                  acc.at[pl.ds(base + nz * ZR, rem)],
                                  sem_z).start()

        for j in range(5):
            idx_start(j)
        idx_wait(0)
        gat_start(0)
        idx_wait(1)
        gat_start(1)

        def z_wait(i, carry):
            pltpu.make_async_copy(zbuf, acc.at[pl.ds(base + i * ZR, ZR)],
                                  sem_z).wait()
            return carry

        lax.fori_loop(0, nz, z_wait, None)
        if rem:
            pltpu.make_async_copy(zbuf.at[pl.ds(0, rem)],
                                  acc.at[pl.ds(base + nz * ZR, rem)],
                                  sem_z).wait()
        plsc.subcore_barrier()

        def body(j, carry):
            @pl.when(j >= 2)
            def _():
                scat_wait(j - 2)

            @pl.when(j + 5 < chunks)
            def _():
                idx_start(j + 5)

            @pl.when(j + 2 < chunks)
            def _():
                idx_wait(j + 2)
                gat_start(j + 2)

            gat_wait(j)
            scat_start(j)
            return carry

        lax.fori_loop(0, chunks, body, None)
        scat_wait(chunks - 2)
        scat_wait(chunks - 1)
        plsc.subcore_barrier()

        pltpu.sync_copy(acc.at[pl.ds(base, rows)],
                        out_hbm.at[cid, pl.ds(base, rows)])

    return agg_kernel(h, src4, dst4)


def _scale_body(x_ref, deg_ref, o_ref):
    deg = deg_ref[:, 0] + deg_ref[:, 1]
    norm = lax.rsqrt(jnp.maximum(deg, 1.0))
    o_ref[...] = x_ref[...] * norm[:, None]


def _out_body(p_ref, deg_ref, w_ref, b_ref, o_ref):
    deg = deg_ref[:, 0] + deg_ref[:, 1]
    norm = lax.rsqrt(jnp.maximum(deg, 1.0))
    rst = (p_ref[0] + p_ref[1]) * norm[:, None]
    o_ref[...] = (jnp.dot(rst, w_ref[...], preferred_element_type=jnp.float32)
                  + b_ref[...])


def kernel(x, edge_index, W, b):
    N, D = x.shape
    E = edge_index.shape[1]
    chunks = E // NW // C
    src2d = edge_index[0].reshape(NW, chunks, C)
    dst2d = edge_index[1].reshape(NW, chunks, C)
    src4 = edge_index[0].reshape(NW, chunks, 1, C)
    dst4 = edge_index[1].reshape(NW, chunks, 1, C)

    deg_s, deg_d = _degrees(src2d, dst2d, E, NP)
    deg_s = deg_s[:, :N].T
    deg_d = deg_d[:, :N].T

    R = 5000
    grid = (N // R,)
    h = pl.pallas_call(
        _scale_body,
        grid=grid,
        in_specs=[
            pl.BlockSpec((R, D), lambda i: (i, 0)),
            pl.BlockSpec((R, NC), lambda i: (i, 0)),
        ],
        out_specs=pl.BlockSpec((R, D), lambda i: (i, 0)),
        out_shape=jax.ShapeDtypeStruct((N, D), jnp.float32),
    )(x, deg_s)

    parts = _aggregate(h, src4, dst4, E, NPA, D)

    out = pl.pallas_call(
        _out_body,
        grid=grid,
        in_specs=[
            pl.BlockSpec((NC, R, D), lambda i: (0, i, 0)),
            pl.BlockSpec((R, NC), lambda i: (i, 0)),
            pl.BlockSpec((D, D), lambda i: (0, 0)),
            pl.BlockSpec((1, D), lambda i: (0, 0)),
        ],
        out_specs=pl.BlockSpec((R, D), lambda i: (i, 0)),
        out_shape=jax.ShapeDtypeStruct((N, D), jnp.float32),
    )(parts, deg_d, W, b[None, :])
    return out
